# Initial kernel scaffold; baseline (speedup 1.0000x reference)
import jax
import jax.numpy as jnp
from jax.experimental import pallas as pl


def kernel(xyz, xy_plane, xz_plane, yz_plane, x_vec, y_vec, z_vec, f_vec):
    raise NotImplementedError("write your pallas kernel here")


# speedup vs baseline: 30.4373x; 30.4373x over previous
import dataclasses
import functools

import jax
import jax.numpy as jnp
from jax import lax
from jax.experimental import pallas as pl
from jax.experimental.pallas import tpu as pltpu
from jax.experimental.pallas import tpu_sc as plsc

XD = 160
RK = 48
FD = 3 * RK
CHO = 27
NPTS = 1048576
NW = 32
PTS_PER_W = NPTS // NW
BB = 128
NBLK = PTS_PER_W // BB


def _sc_feat(xs, ys, zs, t_xy, t_xz, t_yz, v_x, v_y, v_z):
    mesh = plsc.VectorSubcoreMesh(core_axis_name="c", subcore_axis_name="s")
    cp = pltpu.CompilerParams()
    if "needs_layout_passes" in pltpu.CompilerParams.__dataclass_fields__:
        cp = dataclasses.replace(cp, needs_layout_passes=False)
    if "use_tc_tiling_on_sc" in pltpu.CompilerParams.__dataclass_fields__:
        cp = dataclasses.replace(cp, use_tc_tiling_on_sc=False)

    @functools.partial(
        pl.kernel,
        compiler_params=cp,
        out_type=jax.ShapeDtypeStruct((NPTS, FD), jnp.float32),
        mesh=mesh,
        scratch_types=[
            pltpu.VMEM((BB,), jnp.float32),
            pltpu.VMEM((BB,), jnp.float32),
            pltpu.VMEM((BB,), jnp.float32),
            pltpu.VMEM((BB,), jnp.float32),
            pltpu.VMEM((BB,), jnp.float32),
            pltpu.VMEM((BB,), jnp.float32),
            pltpu.VMEM((BB,), jnp.int32),
            pltpu.VMEM((BB,), jnp.int32),
            pltpu.VMEM((BB,), jnp.int32),
            pltpu.VMEM((BB,), jnp.int32),
            pltpu.VMEM((BB,), jnp.int32),
            pltpu.VMEM((BB,), jnp.int32),
            pltpu.VMEM((BB,), jnp.int32),
            pltpu.VMEM((BB,), jnp.int32),
            pltpu.VMEM((BB,), jnp.int32),
            pltpu.VMEM((BB,), jnp.int32),
            pltpu.VMEM((BB, RK), jnp.float32),
            pltpu.VMEM((BB, RK), jnp.float32),
            pltpu.VMEM((BB, RK), jnp.float32),
            pltpu.VMEM((BB, RK), jnp.float32),
            pltpu.VMEM((BB, FD), jnp.float32),
            pltpu.VMEM((XD, RK), jnp.float32),
            pltpu.VMEM((XD, RK), jnp.float32),
            pltpu.VMEM((XD, RK), jnp.float32),
            pltpu.SemaphoreType.DMA,
        ],
    )
    def kern(xs_hbm, ys_hbm, zs_hbm, txy_hbm, txz_hbm, tyz_hbm,
             vx_hbm, vy_hbm, vz_hbm, feat_hbm,
             xv, yv, zv, wxv, wyv, wzv,
             ix0v, ix1v, iy0v, iy1v, iz0v, iz1v,
             c00, c01, c10, c11, r00, r01, r10, r11,
             featv, vxt, vyt, vzt, sem):
        wid = lax.axis_index("s") * 2 + lax.axis_index("c")
        base = wid * PTS_PER_W
        iota = lax.iota(jnp.int32, 16)

        pltpu.sync_copy(vx_hbm, vxt)
        pltpu.sync_copy(vy_hbm, vyt)
        pltpu.sync_copy(vz_hbm, vzt)

        @pl.loop(0, NBLK)
        def _blk(blk):
            off = base + blk * BB
            pltpu.sync_copy(xs_hbm.at[pl.ds(off, BB)], xv)
            pltpu.sync_copy(ys_hbm.at[pl.ds(off, BB)], yv)
            pltpu.sync_copy(zs_hbm.at[pl.ds(off, BB)], zv)

            @pl.loop(0, BB, step=16)
            def _axes(i):
                sl = pl.ds(i, 16)
                for cv, i0v, i1v, wv in ((xv, ix0v, ix1v, wxv),
                                         (yv, iy0v, iy1v, wyv),
                                         (zv, iz0v, iz1v, wzv)):
                    f = (cv[sl] + 1.0) * (0.5 * (XD - 1))
                    f = jnp.minimum(jnp.maximum(f, 0.0), float(XD - 1))
                    i0 = f.astype(jnp.int32)
                    wv[sl] = f - i0.astype(jnp.float32)
                    i0v[sl] = i0
                    i1v[sl] = jnp.minimum(i0 + 1, XD - 1)

            plane_cfg = (
                (txy_hbm, ix0v, ix1v, iy0v, iy1v, wxv, wyv,
                 iz0v, iz1v, wzv, vzt, 0),
                (txz_hbm, ix0v, ix1v, iz0v, iz1v, wxv, wzv,
                 iy0v, iy1v, wyv, vyt, RK),
                (tyz_hbm, iy0v, iy1v, iz0v, iz1v, wyv, wzv,
                 ix0v, ix1v, wxv, vxt, 2 * RK),
            )
            for (pt_hbm, ih0, ih1, iw0, iw1, wh_r, ww_r,
                 jv0, jv1, wv_r, vec_t, fbase) in plane_cfg:

                @pl.loop(0, BB, step=16)
                def _cidx(i, ih0=ih0, ih1=ih1, iw0=iw0, iw1=iw1):
                    sl = pl.ds(i, 16)
                    h0 = ih0[sl] * XD
                    h1 = ih1[sl] * XD
                    w0 = iw0[sl]
                    w1 = iw1[sl]
                    c00[sl] = h0 + w0
                    c01[sl] = h0 + w1
                    c10[sl] = h1 + w0
                    c11[sl] = h1 + w1

                cp0 = pltpu.async_copy(pt_hbm.at[c00], r00, sem)
                cp1 = pltpu.async_copy(pt_hbm.at[c01], r01, sem)
                cp2 = pltpu.async_copy(pt_hbm.at[c10], r10, sem)
                cp3 = pltpu.async_copy(pt_hbm.at[c11], r11, sem)
                cp0.wait()
                cp1.wait()
                cp2.wait()
                cp3.wait()

                @pl.loop(0, BB)
                def _comb(b, wh_r=wh_r, ww_r=ww_r, jv0=jv0, jv1=jv1,
                          wv_r=wv_r, vec_t=vec_t, fbase=fbase):
                    bsel = jnp.zeros((16,), jnp.int32) + b
                    wh = plsc.load_gather(wh_r, [bsel])
                    ww = plsc.load_gather(ww_r, [bsel])
                    wv = plsc.load_gather(wv_r, [bsel])
                    j0 = plsc.load_gather(jv0, [bsel])
                    j1 = plsc.load_gather(jv1, [bsel])
                    mh = 1.0 - wh
                    mw = 1.0 - ww
                    mv = 1.0 - wv
                    w00 = mh * mw
                    w01 = mh * ww
                    w10 = wh * mw
                    w11 = wh * ww
                    for k in range(RK // 16):
                        sl = pl.ds(k * 16, 16)
                        col = iota + (k * 16)
                        acc = (r00[b, sl] * w00 + r01[b, sl] * w01
                               + r10[b, sl] * w10 + r11[b, sl] * w11)
                        u0 = plsc.load_gather(vec_t, [j0, col])
                        u1 = plsc.load_gather(vec_t, [j1, col])
                        featv[b, pl.ds(fbase + k * 16, 16)] = (
                            acc * (mv * u0 + wv * u1))

            pltpu.sync_copy(featv, feat_hbm.at[pl.ds(off, BB)])

    return kern(xs, ys, zs, t_xy, t_xz, t_yz, v_x, v_y, v_z)


def _tc_matmul(feat, f_vec):
    bm = 2048

    def mm(x_ref, w_ref, o_ref):
        o_ref[...] = jnp.dot(x_ref[...], w_ref[...],
                             preferred_element_type=jnp.float32)

    return pl.pallas_call(
        mm,
        grid=(NPTS // bm,),
        in_specs=[pl.BlockSpec((bm, FD), lambda i: (i, 0)),
                  pl.BlockSpec((FD, CHO), lambda i: (0, 0))],
        out_specs=pl.BlockSpec((bm, CHO), lambda i: (i, 0)),
        out_shape=jax.ShapeDtypeStruct((NPTS, CHO), jnp.float32),
    )(feat, f_vec)


def kernel(xyz, xy_plane, xz_plane, yz_plane, x_vec, y_vec, z_vec, f_vec):
    xs = xyz[:, 0] + 0.0
    ys = xyz[:, 1] + 0.0
    zs = xyz[:, 2] + 0.0
    t_xy = xy_plane[0].transpose(1, 2, 0).reshape(XD * XD, RK)
    t_xz = xz_plane[0].transpose(1, 2, 0).reshape(XD * XD, RK)
    t_yz = yz_plane[0].transpose(1, 2, 0).reshape(XD * XD, RK)
    v_x = x_vec[0, :, :, 0].T
    v_y = y_vec[0, :, :, 0].T
    v_z = z_vec[0, :, :, 0].T
    feat = _sc_feat(xs, ys, zs, t_xy, t_xz, t_yz, v_x, v_y, v_z)
    return _tc_matmul(feat, f_vec)
